# baseline (speedup 1.0000x reference)
import functools
import jax
import jax.numpy as jnp
from jax import lax
from jax.experimental import pallas as pl
from jax.experimental.pallas import tpu as pltpu
from jax.experimental.pallas import tpu_sc as plsc

N_NODES = 10000
N_EDGES = 320000
D_FEAT = 128
D_EDGE = 16
HALF = 64
N_GRAPHS = 256
EPS = 1e-7

NCORE = 2
NSUB = 16
CHUNK = 80
EDGES_PER_TILE = N_EDGES // NSUB
NCHUNKS = EDGES_PER_TILE // CHUNK
SUPER = 50
ROWS_PER_TILE = 624
ROWS_TAIL = N_NODES - NSUB * ROWS_PER_TILE


def _ea_body(ea_ref, w_ref, b_ref, o0_ref, o1_ref, o2_ref):
    prod = jnp.dot(ea_ref[...], w_ref[...],
                   preferred_element_type=jnp.float32) + b_ref[...]
    for l, o_ref in enumerate((o0_ref, o1_ref, o2_ref)):
        o_ref[0] = prod[:, l * 128:l * 128 + 64]
        o_ref[1] = prod[:, l * 128 + 64:(l + 1) * 128]


def _edge_linears(edge_attr, wcat, bcat):
    be = 2000
    grid = (N_EDGES // be,)
    out_sds = jax.ShapeDtypeStruct((NCORE, N_EDGES, HALF), jnp.float32)
    return pl.pallas_call(
        _ea_body,
        grid=grid,
        in_specs=[
            pl.BlockSpec((be, D_EDGE), lambda e: (e, 0)),
            pl.BlockSpec((D_EDGE, 3 * D_FEAT), lambda e: (0, 0)),
            pl.BlockSpec((1, 3 * D_FEAT), lambda e: (0, 0)),
        ],
        out_specs=[pl.BlockSpec((NCORE, be, HALF), lambda e: (0, e, 0))] * 3,
        out_shape=[out_sds] * 3,
    )(edge_attr, wcat, bcat)


def _sc_edge_body(xa_hbm, xb_hbm, eas_hbm, pack_hbm, zeros_hbm, out_hbm,
                  acc, slab, xr0, xr1, ea0, ea1, ob0, ob1,
                  ea_sem, g_sem, sc_sem):
    c = lax.axis_index("c")
    s = lax.axis_index("s")

    r0 = s * ROWS_PER_TILE
    pltpu.sync_copy(zeros_hbm.at[pl.ds(0, ROWS_PER_TILE)],
                    acc.at[pl.ds(r0, ROWS_PER_TILE)])

    @pl.when(s == NSUB - 1)
    def _():
        pltpu.sync_copy(zeros_hbm.at[pl.ds(0, ROWS_TAIL)],
                        acc.at[pl.ds(NSUB * ROWS_PER_TILE, ROWS_TAIL)])

    plsc.subcore_barrier()

    ea_off = c * N_EDGES
    cid0 = s * NCHUNKS

    def wait_gather(xr):
        pltpu.make_async_copy(xa_hbm.at[slab.at[0, 0]], xr, g_sem).wait()

    def wait_ea(ea):
        pltpu.make_async_copy(eas_hbm.at[pl.ds(0, CHUNK)], ea, ea_sem).wait()

    def wait_scatter(ob):
        pltpu.make_async_copy(ob, acc.at[slab.at[0, 1]], sc_sem).wait()

    def super_blk(s_i, _):
        sup_base = cid0 + s_i * SUPER

        pltpu.sync_copy(pack_hbm.at[pl.ds(sup_base, SUPER)], slab)

        def issue_chunk(j, xr, ea):
            @pl.when(c == 0)
            def _():
                pltpu.async_copy(xa_hbm.at[slab.at[j, 0]], xr, g_sem)

            @pl.when(c == 1)
            def _():
                pltpu.async_copy(xb_hbm.at[slab.at[j, 0]], xr, g_sem)

            pltpu.async_copy(
                eas_hbm.at[pl.ds(ea_off + (sup_base + j) * CHUNK, CHUNK)],
                ea, ea_sem)

        def phase(j, cur_xr, cur_ea, cur_ob, nxt_xr, nxt_ea, drain_scatter):
            jnext = jnp.where(j + 1 == SUPER, 0, j + 1)
            issue_chunk(jnext, nxt_xr, nxt_ea)
            wait_gather(cur_xr)
            wait_ea(cur_ea)

            @pl.when(drain_scatter)
            def _():
                wait_scatter(cur_ob)

            for e in range(CHUNK):
                for k in range(HALF // 16):
                    xv = cur_xr[e, pl.ds(16 * k, 16)]
                    ev = cur_ea[e, pl.ds(16 * k, 16)]
                    m = jnp.maximum(xv + ev, EPS)
                    t = jnp.exp(m)
                    cur_ob[e, pl.ds(16 * k, 16)] = t * m
                    cur_ob[e, pl.ds(HALF + 16 * k, 16)] = t
            pltpu.async_copy(cur_ob, acc.at[slab.at[j, 1]], sc_sem, add=True)

        issue_chunk(jnp.int32(0), xr0, ea0)

        def step(j2, _):
            j = 2 * j2
            phase(j, xr0, ea0, ob0, xr1, ea1, j2 > 0)
            phase(j + 1, xr1, ea1, ob1, xr0, ea0, j2 > 0)
            return 0

        lax.fori_loop(0, SUPER // 2, step, 0)

        wait_gather(xr0)
        wait_ea(ea0)
        wait_scatter(ob0)
        wait_scatter(ob1)
        return 0

    lax.fori_loop(0, NCHUNKS // SUPER, super_blk, 0)

    plsc.subcore_barrier()

    pltpu.sync_copy(acc.at[pl.ds(r0, ROWS_PER_TILE)],
                    out_hbm.at[c, pl.ds(r0, ROWS_PER_TILE)])

    @pl.when(s == NSUB - 1)
    def _():
        pltpu.sync_copy(acc.at[pl.ds(NSUB * ROWS_PER_TILE, ROWS_TAIL)],
                        out_hbm.at[c, pl.ds(NSUB * ROWS_PER_TILE, ROWS_TAIL)])


def _sc_edge_pass(xa, xb, eas, pack, zeros):
    mesh = plsc.VectorSubcoreMesh(core_axis_name="c", subcore_axis_name="s")
    f = pl.kernel(
        _sc_edge_body,
        out_type=jax.ShapeDtypeStruct((NCORE, N_NODES, 2 * HALF), jnp.float32),
        mesh=mesh,
        compiler_params=pltpu.CompilerParams(
            use_tc_tiling_on_sc=False,
            disable_bounds_checks=True,
            disable_semaphore_checks=True,
        ),
        scratch_types=[
            pltpu.VMEM_SHARED((N_NODES, 2 * HALF), jnp.float32),
            pltpu.VMEM((SUPER, 2, CHUNK), jnp.int32),
            pltpu.VMEM((CHUNK, HALF), jnp.float32),
            pltpu.VMEM((CHUNK, HALF), jnp.float32),
            pltpu.VMEM((CHUNK, HALF), jnp.float32),
            pltpu.VMEM((CHUNK, HALF), jnp.float32),
            pltpu.VMEM((CHUNK, 2 * HALF), jnp.float32),
            pltpu.VMEM((CHUNK, 2 * HALF), jnp.float32),
            pltpu.SemaphoreType.DMA,
            pltpu.SemaphoreType.DMA,
            pltpu.SemaphoreType.DMA,
        ],
    )
    return f(xa, xb, eas, pack, zeros)


def _update_body(sc_ref, xs_ref, w1_ref, b1_ref, w2_ref, b2_ref,
                 lw_ref, lb_ref, o_ref):
    num = jnp.concatenate([sc_ref[0, :, :HALF], sc_ref[1, :, :HALF]], axis=1)
    den = jnp.concatenate([sc_ref[0, :, HALF:], sc_ref[1, :, HALF:]], axis=1)
    x = jnp.concatenate([xs_ref[0], xs_ref[1]], axis=1)
    h = num / (den + 1e-16) + x
    h = jnp.maximum(jnp.dot(h, w1_ref[...],
                            preferred_element_type=jnp.float32) + b1_ref[...],
                    0.0)
    h = jnp.dot(h, w2_ref[...], preferred_element_type=jnp.float32) + b2_ref[...]
    y = jnp.maximum(jnp.dot(h, lw_ref[...],
                            preferred_element_type=jnp.float32) + lb_ref[...],
                    0.0)
    o_ref[0] = y[:, :HALF]
    o_ref[1] = y[:, HALF:]


def _node_update(sc_out, xs, gp, lp):
    bn = 2000
    grid = (N_NODES // bn,)
    full = lambda a: pl.BlockSpec(a.shape, lambda n: (0,) * a.ndim)
    w1 = gp["W1"]
    b1 = gp["b1"].reshape(1, -1)
    w2 = gp["W2"]
    b2 = gp["b2"].reshape(1, -1)
    lw = lp["W"]
    lb = lp["b"].reshape(1, -1)
    return pl.pallas_call(
        _update_body,
        grid=grid,
        in_specs=[
            pl.BlockSpec((NCORE, bn, 2 * HALF), lambda n: (0, n, 0)),
            pl.BlockSpec((NCORE, bn, HALF), lambda n: (0, n, 0)),
            full(w1), full(b1), full(w2), full(b2), full(lw), full(lb),
        ],
        out_specs=pl.BlockSpec((NCORE, bn, HALF), lambda n: (0, n, 0)),
        out_shape=jax.ShapeDtypeStruct((NCORE, N_NODES, HALF), jnp.float32),
    )(sc_out, xs, w1, b1, w2, b2, lw, lb)


def _update_pool_body(sc_ref, xs_ref, w1_ref, b1_ref, w2_ref, b2_ref,
                      lw_ref, lb_ref, bi_ref, sums_ref, cnt_ref):
    @pl.when(pl.program_id(0) == 0)
    def _():
        sums_ref[...] = jnp.zeros_like(sums_ref)
        cnt_ref[...] = jnp.zeros_like(cnt_ref)

    num = jnp.concatenate([sc_ref[0, :, :HALF], sc_ref[1, :, :HALF]], axis=1)
    den = jnp.concatenate([sc_ref[0, :, HALF:], sc_ref[1, :, HALF:]], axis=1)
    x = jnp.concatenate([xs_ref[0], xs_ref[1]], axis=1)
    h = num / (den + 1e-16) + x
    h = jnp.maximum(jnp.dot(h, w1_ref[...],
                            preferred_element_type=jnp.float32) + b1_ref[...],
                    0.0)
    h = jnp.dot(h, w2_ref[...], preferred_element_type=jnp.float32) + b2_ref[...]
    y = jnp.maximum(jnp.dot(h, lw_ref[...],
                            preferred_element_type=jnp.float32) + lb_ref[...],
                    0.0)
    bi = bi_ref[0, 0, :]
    gids = lax.broadcasted_iota(jnp.int32, (N_GRAPHS, bi.shape[0]), 0)
    mask = (bi[None, :] == gids).astype(jnp.float32)
    sums_ref[...] += jnp.dot(mask, y, preferred_element_type=jnp.float32)
    cnt_ref[...] += jnp.sum(mask, axis=1, keepdims=True)


def _update_pool(sc_out, xs, gp, lp, batch_index):
    bn = 2000
    grid = (N_NODES // bn,)
    full = lambda a: pl.BlockSpec(a.shape, lambda n: (0,) * a.ndim)
    w1 = gp["W1"]
    b1 = gp["b1"].reshape(1, -1)
    w2 = gp["W2"]
    b2 = gp["b2"].reshape(1, -1)
    lw = lp["W"]
    lb = lp["b"].reshape(1, -1)
    bi3 = batch_index.reshape(N_NODES // bn, 1, bn)
    return pl.pallas_call(
        _update_pool_body,
        grid=grid,
        in_specs=[
            pl.BlockSpec((NCORE, bn, 2 * HALF), lambda n: (0, n, 0)),
            pl.BlockSpec((NCORE, bn, HALF), lambda n: (0, n, 0)),
            full(w1), full(b1), full(w2), full(b2), full(lw), full(lb),
            pl.BlockSpec((1, 1, bn), lambda n: (n, 0, 0)),
        ],
        out_specs=[
            pl.BlockSpec((N_GRAPHS, D_FEAT), lambda n: (0, 0)),
            pl.BlockSpec((N_GRAPHS, 1), lambda n: (0, 0)),
        ],
        out_shape=[
            jax.ShapeDtypeStruct((N_GRAPHS, D_FEAT), jnp.float32),
            jax.ShapeDtypeStruct((N_GRAPHS, 1), jnp.float32),
        ],
    )(sc_out, xs, w1, b1, w2, b2, lw, lb, bi3)


def _head_body(sums_ref, cnt_ref, mol_ref,
               m0w, m0b, m1w, m1b, m2w, m2b,
               p0w, p0b, p1w, p1b, ow, ob, o_ref):
    h1 = sums_ref[...] / jnp.maximum(cnt_ref[...], 1.0)
    h2 = mol_ref[...]
    for w, b in ((m0w, m0b), (m1w, m1b), (m2w, m2b)):
        h2 = jnp.maximum(jnp.dot(h2, w[...],
                                 preferred_element_type=jnp.float32) + b[...],
                         0.0)
    w0 = p0w[...]
    h = jnp.maximum(jnp.dot(h1, w0[:D_FEAT],
                            preferred_element_type=jnp.float32)
                    + jnp.dot(h2, w0[D_FEAT:],
                              preferred_element_type=jnp.float32)
                    + p0b[...], 0.0)
    h = jnp.maximum(jnp.dot(h, p1w[...],
                            preferred_element_type=jnp.float32) + p1b[...],
                    0.0)
    o_ref[...] = jnp.dot(h, ow[...],
                         preferred_element_type=jnp.float32) + ob[...]


def _head(sums, cnt, mol_features, params):
    mlp = params["mlp"]
    pred = params["pred"]
    out = params["out"]
    args = [sums, cnt, mol_features,
            mlp[0]["W"], mlp[0]["b"].reshape(1, -1),
            mlp[1]["W"], mlp[1]["b"].reshape(1, -1),
            mlp[2]["W"], mlp[2]["b"].reshape(1, -1),
            pred[0]["W"], pred[0]["b"].reshape(1, -1),
            pred[1]["W"], pred[1]["b"].reshape(1, -1),
            out["W"], out["b"].reshape(1, -1)]
    full = lambda a: pl.BlockSpec(a.shape, lambda: (0,) * a.ndim)
    return pl.pallas_call(
        _head_body,
        in_specs=[full(a) for a in args],
        out_specs=pl.BlockSpec((N_GRAPHS, 1), lambda: (0, 0)),
        out_shape=jax.ShapeDtypeStruct((N_GRAPHS, 1), jnp.float32),
    )(*args)


def kernel(x, edge_index, edge_attr, batch_index, mol_features, params):
    pack = jnp.concatenate(
        [edge_index[0].reshape(N_EDGES // CHUNK, 1, CHUNK),
         edge_index[1].reshape(N_EDGES // CHUNK, 1, CHUNK)], axis=1)

    wcat = jnp.concatenate([p["We"] for p in params["gcn"]], axis=1)
    bcat = jnp.concatenate([p["be"] for p in params["gcn"]]).reshape(1, -1)
    eas = _edge_linears(edge_attr, wcat, bcat + EPS)

    zeros = jnp.zeros((ROWS_PER_TILE, 2 * HALF), jnp.float32)
    xs = jnp.stack([x[:, :HALF], x[:, HALF:]])
    sc1 = _sc_edge_pass(xs[0], xs[1],
                        eas[0].reshape(NCORE * N_EDGES, HALF), pack, zeros)
    xs1 = _node_update(sc1, xs, params["gcn"][0], params["lin"][0])
    sc2 = _sc_edge_pass(xs1[0], xs1[1],
                        eas[1].reshape(NCORE * N_EDGES, HALF), pack, zeros)
    xs2 = _node_update(sc2, xs1, params["gcn"][1], params["lin"][1])
    sc3 = _sc_edge_pass(xs2[0], xs2[1],
                        eas[2].reshape(NCORE * N_EDGES, HALF), pack, zeros)
    sums, cnt = _update_pool(sc3, xs2, params["gcn"][2], params["lin"][2],
                             batch_index)
    return _head(sums, cnt, mol_features, params)


# speedup vs baseline: 1.0011x; 1.0011x over previous
import functools
import jax
import jax.numpy as jnp
from jax import lax
from jax.experimental import pallas as pl
from jax.experimental.pallas import tpu as pltpu
from jax.experimental.pallas import tpu_sc as plsc

N_NODES = 10000
N_EDGES = 320000
D_FEAT = 128
D_EDGE = 16
HALF = 64
N_GRAPHS = 256
EPS = 1e-7

NCORE = 2
NSUB = 16
CHUNK = 80
EDGES_PER_TILE = N_EDGES // NSUB
NCHUNKS = EDGES_PER_TILE // CHUNK
SUPER = 50
ROWS_PER_TILE = 624
ROWS_TAIL = N_NODES - NSUB * ROWS_PER_TILE


def _ea_body(ea_ref, w_ref, b_ref, o0_ref, o1_ref, o2_ref):
    prod = jnp.dot(ea_ref[...], w_ref[...],
                   preferred_element_type=jnp.float32) + b_ref[...]
    for l, o_ref in enumerate((o0_ref, o1_ref, o2_ref)):
        o_ref[0] = prod[:, l * 128:l * 128 + 64]
        o_ref[1] = prod[:, l * 128 + 64:(l + 1) * 128]


def _edge_linears(edge_attr, wcat, bcat):
    be = 2000
    grid = (N_EDGES // be,)
    out_sds = jax.ShapeDtypeStruct((NCORE, N_EDGES, HALF), jnp.float32)
    return pl.pallas_call(
        _ea_body,
        grid=grid,
        in_specs=[
            pl.BlockSpec((be, D_EDGE), lambda e: (e, 0)),
            pl.BlockSpec((D_EDGE, 3 * D_FEAT), lambda e: (0, 0)),
            pl.BlockSpec((1, 3 * D_FEAT), lambda e: (0, 0)),
        ],
        out_specs=[pl.BlockSpec((NCORE, be, HALF), lambda e: (0, e, 0))] * 3,
        out_shape=[out_sds] * 3,
    )(edge_attr, wcat, bcat)


def _sc_edge_body(xa_hbm, xb_hbm, eas_hbm, pack_hbm, zeros_hbm, out_hbm,
                  acc, slab, xr0, xr1, ea0, ea1, ob0, ob1,
                  ea_sem, g_sem, sc_sem):
    c = lax.axis_index("c")
    s = lax.axis_index("s")

    r0 = s * ROWS_PER_TILE
    pltpu.sync_copy(zeros_hbm.at[pl.ds(0, ROWS_PER_TILE)],
                    acc.at[pl.ds(r0, ROWS_PER_TILE)])

    @pl.when(s == NSUB - 1)
    def _():
        pltpu.sync_copy(zeros_hbm.at[pl.ds(0, ROWS_TAIL)],
                        acc.at[pl.ds(NSUB * ROWS_PER_TILE, ROWS_TAIL)])

    plsc.subcore_barrier()

    cid0 = s * NCHUNKS

    def wait_gather(xr):
        pltpu.make_async_copy(xa_hbm.at[slab.at[0, 0]], xr, g_sem).wait()

    def wait_ea(ea):
        pltpu.make_async_copy(eas_hbm.at[0, pl.ds(0, CHUNK)], ea,
                              ea_sem).wait()

    def wait_scatter(ob):
        pltpu.make_async_copy(ob, acc.at[slab.at[0, 1]], sc_sem).wait()

    def super_blk(s_i, _):
        sup_base = cid0 + s_i * SUPER

        pltpu.sync_copy(pack_hbm.at[pl.ds(sup_base, SUPER)], slab)

        def issue_chunk(j, xr, ea):
            @pl.when(c == 0)
            def _():
                pltpu.async_copy(xa_hbm.at[slab.at[j, 0]], xr, g_sem)

            @pl.when(c == 1)
            def _():
                pltpu.async_copy(xb_hbm.at[slab.at[j, 0]], xr, g_sem)

            pltpu.async_copy(
                eas_hbm.at[c, pl.ds((sup_base + j) * CHUNK, CHUNK)],
                ea, ea_sem)

        def phase(j, cur_xr, cur_ea, cur_ob, nxt_xr, nxt_ea, drain_scatter):
            jnext = jnp.where(j + 1 == SUPER, 0, j + 1)
            issue_chunk(jnext, nxt_xr, nxt_ea)
            wait_gather(cur_xr)
            wait_ea(cur_ea)

            @pl.when(drain_scatter)
            def _():
                wait_scatter(cur_ob)

            for e in range(CHUNK):
                for k in range(HALF // 16):
                    xv = cur_xr[e, pl.ds(16 * k, 16)]
                    ev = cur_ea[e, pl.ds(16 * k, 16)]
                    m = jnp.maximum(xv + ev, EPS)
                    t = jnp.exp(m)
                    cur_ob[e, pl.ds(16 * k, 16)] = t * m
                    cur_ob[e, pl.ds(HALF + 16 * k, 16)] = t
            pltpu.async_copy(cur_ob, acc.at[slab.at[j, 1]], sc_sem, add=True)

        issue_chunk(jnp.int32(0), xr0, ea0)

        def step(j2, _):
            j = 2 * j2
            phase(j, xr0, ea0, ob0, xr1, ea1, j2 > 0)
            phase(j + 1, xr1, ea1, ob1, xr0, ea0, j2 > 0)
            return 0

        lax.fori_loop(0, SUPER // 2, step, 0)

        wait_gather(xr0)
        wait_ea(ea0)
        wait_scatter(ob0)
        wait_scatter(ob1)
        return 0

    lax.fori_loop(0, NCHUNKS // SUPER, super_blk, 0)

    plsc.subcore_barrier()

    pltpu.sync_copy(acc.at[pl.ds(r0, ROWS_PER_TILE)],
                    out_hbm.at[c, pl.ds(r0, ROWS_PER_TILE)])

    @pl.when(s == NSUB - 1)
    def _():
        pltpu.sync_copy(acc.at[pl.ds(NSUB * ROWS_PER_TILE, ROWS_TAIL)],
                        out_hbm.at[c, pl.ds(NSUB * ROWS_PER_TILE, ROWS_TAIL)])


def _sc_edge_pass(xa, xb, eas, pack, zeros):
    mesh = plsc.VectorSubcoreMesh(core_axis_name="c", subcore_axis_name="s")
    f = pl.kernel(
        _sc_edge_body,
        out_type=jax.ShapeDtypeStruct((NCORE, N_NODES, 2 * HALF), jnp.float32),
        mesh=mesh,
        compiler_params=pltpu.CompilerParams(use_tc_tiling_on_sc=False),
        scratch_types=[
            pltpu.VMEM_SHARED((N_NODES, 2 * HALF), jnp.float32),
            pltpu.VMEM((SUPER, 2, CHUNK), jnp.int32),
            pltpu.VMEM((CHUNK, HALF), jnp.float32),
            pltpu.VMEM((CHUNK, HALF), jnp.float32),
            pltpu.VMEM((CHUNK, HALF), jnp.float32),
            pltpu.VMEM((CHUNK, HALF), jnp.float32),
            pltpu.VMEM((CHUNK, 2 * HALF), jnp.float32),
            pltpu.VMEM((CHUNK, 2 * HALF), jnp.float32),
            pltpu.SemaphoreType.DMA,
            pltpu.SemaphoreType.DMA,
            pltpu.SemaphoreType.DMA,
        ],
    )
    return f(xa, xb, eas, pack, zeros)


def _update_body(sc_ref, xs_ref, w1_ref, b1_ref, w2_ref, b2_ref,
                 lw_ref, lb_ref, o_ref):
    num = jnp.concatenate([sc_ref[0, :, :HALF], sc_ref[1, :, :HALF]], axis=1)
    den = jnp.concatenate([sc_ref[0, :, HALF:], sc_ref[1, :, HALF:]], axis=1)
    x = jnp.concatenate([xs_ref[0], xs_ref[1]], axis=1)
    h = num / (den + 1e-16) + x
    h = jnp.maximum(jnp.dot(h, w1_ref[...],
                            preferred_element_type=jnp.float32) + b1_ref[...],
                    0.0)
    h = jnp.dot(h, w2_ref[...], preferred_element_type=jnp.float32) + b2_ref[...]
    y = jnp.maximum(jnp.dot(h, lw_ref[...],
                            preferred_element_type=jnp.float32) + lb_ref[...],
                    0.0)
    o_ref[0] = y[:, :HALF]
    o_ref[1] = y[:, HALF:]


def _node_update(sc_out, xs, gp, lp):
    bn = 2000
    grid = (N_NODES // bn,)
    full = lambda a: pl.BlockSpec(a.shape, lambda n: (0,) * a.ndim)
    w1 = gp["W1"]
    b1 = gp["b1"].reshape(1, -1)
    w2 = gp["W2"]
    b2 = gp["b2"].reshape(1, -1)
    lw = lp["W"]
    lb = lp["b"].reshape(1, -1)
    return pl.pallas_call(
        _update_body,
        grid=grid,
        in_specs=[
            pl.BlockSpec((NCORE, bn, 2 * HALF), lambda n: (0, n, 0)),
            pl.BlockSpec((NCORE, bn, HALF), lambda n: (0, n, 0)),
            full(w1), full(b1), full(w2), full(b2), full(lw), full(lb),
        ],
        out_specs=pl.BlockSpec((NCORE, bn, HALF), lambda n: (0, n, 0)),
        out_shape=jax.ShapeDtypeStruct((NCORE, N_NODES, HALF), jnp.float32),
    )(sc_out, xs, w1, b1, w2, b2, lw, lb)


def _update_pool_body(sc_ref, xs_ref, w1_ref, b1_ref, w2_ref, b2_ref,
                      lw_ref, lb_ref, bi_ref, sums_ref, cnt_ref):
    @pl.when(pl.program_id(0) == 0)
    def _():
        sums_ref[...] = jnp.zeros_like(sums_ref)
        cnt_ref[...] = jnp.zeros_like(cnt_ref)

    num = jnp.concatenate([sc_ref[0, :, :HALF], sc_ref[1, :, :HALF]], axis=1)
    den = jnp.concatenate([sc_ref[0, :, HALF:], sc_ref[1, :, HALF:]], axis=1)
    x = jnp.concatenate([xs_ref[0], xs_ref[1]], axis=1)
    h = num / (den + 1e-16) + x
    h = jnp.maximum(jnp.dot(h, w1_ref[...],
                            preferred_element_type=jnp.float32) + b1_ref[...],
                    0.0)
    h = jnp.dot(h, w2_ref[...], preferred_element_type=jnp.float32) + b2_ref[...]
    y = jnp.maximum(jnp.dot(h, lw_ref[...],
                            preferred_element_type=jnp.float32) + lb_ref[...],
                    0.0)
    bi = bi_ref[0, 0, :]
    gids = lax.broadcasted_iota(jnp.int32, (N_GRAPHS, bi.shape[0]), 0)
    mask = (bi[None, :] == gids).astype(jnp.float32)
    sums_ref[...] += jnp.dot(mask, y, preferred_element_type=jnp.float32)
    cnt_ref[...] += jnp.sum(mask, axis=1, keepdims=True)


def _update_pool(sc_out, xs, gp, lp, batch_index):
    bn = 2000
    grid = (N_NODES // bn,)
    full = lambda a: pl.BlockSpec(a.shape, lambda n: (0,) * a.ndim)
    w1 = gp["W1"]
    b1 = gp["b1"].reshape(1, -1)
    w2 = gp["W2"]
    b2 = gp["b2"].reshape(1, -1)
    lw = lp["W"]
    lb = lp["b"].reshape(1, -1)
    bi3 = batch_index.reshape(N_NODES // bn, 1, bn)
    return pl.pallas_call(
        _update_pool_body,
        grid=grid,
        in_specs=[
            pl.BlockSpec((NCORE, bn, 2 * HALF), lambda n: (0, n, 0)),
            pl.BlockSpec((NCORE, bn, HALF), lambda n: (0, n, 0)),
            full(w1), full(b1), full(w2), full(b2), full(lw), full(lb),
            pl.BlockSpec((1, 1, bn), lambda n: (n, 0, 0)),
        ],
        out_specs=[
            pl.BlockSpec((N_GRAPHS, D_FEAT), lambda n: (0, 0)),
            pl.BlockSpec((N_GRAPHS, 1), lambda n: (0, 0)),
        ],
        out_shape=[
            jax.ShapeDtypeStruct((N_GRAPHS, D_FEAT), jnp.float32),
            jax.ShapeDtypeStruct((N_GRAPHS, 1), jnp.float32),
        ],
    )(sc_out, xs, w1, b1, w2, b2, lw, lb, bi3)


def _head_body(sums_ref, cnt_ref, mol_ref,
               m0w, m0b, m1w, m1b, m2w, m2b,
               p0w, p0b, p1w, p1b, ow, ob, o_ref):
    h1 = sums_ref[...] / jnp.maximum(cnt_ref[...], 1.0)
    h2 = mol_ref[...]
    for w, b in ((m0w, m0b), (m1w, m1b), (m2w, m2b)):
        h2 = jnp.maximum(jnp.dot(h2, w[...],
                                 preferred_element_type=jnp.float32) + b[...],
                         0.0)
    w0 = p0w[...]
    h = jnp.maximum(jnp.dot(h1, w0[:D_FEAT],
                            preferred_element_type=jnp.float32)
                    + jnp.dot(h2, w0[D_FEAT:],
                              preferred_element_type=jnp.float32)
                    + p0b[...], 0.0)
    h = jnp.maximum(jnp.dot(h, p1w[...],
                            preferred_element_type=jnp.float32) + p1b[...],
                    0.0)
    o_ref[...] = jnp.dot(h, ow[...],
                         preferred_element_type=jnp.float32) + ob[...]


def _head(sums, cnt, mol_features, params):
    mlp = params["mlp"]
    pred = params["pred"]
    out = params["out"]
    args = [sums, cnt, mol_features,
            mlp[0]["W"], mlp[0]["b"].reshape(1, -1),
            mlp[1]["W"], mlp[1]["b"].reshape(1, -1),
            mlp[2]["W"], mlp[2]["b"].reshape(1, -1),
            pred[0]["W"], pred[0]["b"].reshape(1, -1),
            pred[1]["W"], pred[1]["b"].reshape(1, -1),
            out["W"], out["b"].reshape(1, -1)]
    full = lambda a: pl.BlockSpec(a.shape, lambda: (0,) * a.ndim)
    return pl.pallas_call(
        _head_body,
        in_specs=[full(a) for a in args],
        out_specs=pl.BlockSpec((N_GRAPHS, 1), lambda: (0, 0)),
        out_shape=jax.ShapeDtypeStruct((N_GRAPHS, 1), jnp.float32),
    )(*args)


def kernel(x, edge_index, edge_attr, batch_index, mol_features, params):
    pack = jnp.concatenate(
        [edge_index[0].reshape(N_EDGES // CHUNK, 1, CHUNK),
         edge_index[1].reshape(N_EDGES // CHUNK, 1, CHUNK)], axis=1)

    wcat = jnp.concatenate([p["We"] for p in params["gcn"]], axis=1)
    bcat = jnp.concatenate([p["be"] for p in params["gcn"]]).reshape(1, -1)
    eas = _edge_linears(edge_attr, wcat, bcat + EPS)

    zeros = jnp.zeros((ROWS_PER_TILE, 2 * HALF), jnp.float32)
    xs = jnp.stack([x[:, :HALF], x[:, HALF:]])
    sc1 = _sc_edge_pass(xs[0], xs[1], eas[0], pack, zeros)
    xs1 = _node_update(sc1, xs, params["gcn"][0], params["lin"][0])
    sc2 = _sc_edge_pass(xs1[0], xs1[1], eas[1], pack, zeros)
    xs2 = _node_update(sc2, xs1, params["gcn"][1], params["lin"][1])
    sc3 = _sc_edge_pass(xs2[0], xs2[1], eas[2], pack, zeros)
    sums, cnt = _update_pool(sc3, xs2, params["gcn"][2], params["lin"][2],
                             batch_index)
    return _head(sums, cnt, mol_features, params)


# speedup vs baseline: 1.0346x; 1.0335x over previous
import functools
import jax
import jax.numpy as jnp
from jax import lax
from jax.experimental import pallas as pl
from jax.experimental.pallas import tpu as pltpu
from jax.experimental.pallas import tpu_sc as plsc

N_NODES = 10000
N_EDGES = 320000
D_FEAT = 128
D_EDGE = 16
HALF = 64
N_GRAPHS = 256
EPS = 1e-7

NCORE = 2
NSUB = 16
CHUNK = 80
EDGES_PER_TILE = N_EDGES // NSUB
NCHUNKS = EDGES_PER_TILE // CHUNK
SUPER = 50
ROWS_PER_TILE = 624
ROWS_TAIL = N_NODES - NSUB * ROWS_PER_TILE


def _ea_body(ea_ref, w_ref, b_ref, o0_ref, o1_ref, o2_ref):
    prod = (jnp.dot(ea_ref[...], w_ref[...],
                    preferred_element_type=jnp.float32)
            + b_ref[...]).astype(jnp.bfloat16)
    for l, o_ref in enumerate((o0_ref, o1_ref, o2_ref)):
        o_ref[0] = prod[:, l * 128:l * 128 + 64]
        o_ref[1] = prod[:, l * 128 + 64:(l + 1) * 128]


def _edge_linears(edge_attr, wcat, bcat):
    be = 2000
    grid = (N_EDGES // be,)
    out_sds = jax.ShapeDtypeStruct((NCORE, N_EDGES, HALF), jnp.bfloat16)
    return pl.pallas_call(
        _ea_body,
        grid=grid,
        in_specs=[
            pl.BlockSpec((be, D_EDGE), lambda e: (e, 0)),
            pl.BlockSpec((D_EDGE, 3 * D_FEAT), lambda e: (0, 0)),
            pl.BlockSpec((1, 3 * D_FEAT), lambda e: (0, 0)),
        ],
        out_specs=[pl.BlockSpec((NCORE, be, HALF), lambda e: (0, e, 0))] * 3,
        out_shape=[out_sds] * 3,
    )(edge_attr, wcat, bcat)


def _sc_edge_body(xa_hbm, xb_hbm, eas_hbm, pack_hbm, zeros_hbm, out_hbm,
                  acc, slab, xr0, xr1, ea0, ea1, ob0, ob1,
                  ea_sem, g_sem, sc_sem):
    c = lax.axis_index("c")
    s = lax.axis_index("s")

    r0 = s * ROWS_PER_TILE
    pltpu.sync_copy(zeros_hbm.at[pl.ds(0, ROWS_PER_TILE)],
                    acc.at[pl.ds(r0, ROWS_PER_TILE)])

    @pl.when(s == NSUB - 1)
    def _():
        pltpu.sync_copy(zeros_hbm.at[pl.ds(0, ROWS_TAIL)],
                        acc.at[pl.ds(NSUB * ROWS_PER_TILE, ROWS_TAIL)])

    plsc.subcore_barrier()

    cid0 = s * NCHUNKS

    def wait_gather(xr):
        pltpu.make_async_copy(xa_hbm.at[slab.at[0, 0]], xr, g_sem).wait()

    def wait_ea(ea):
        pltpu.make_async_copy(eas_hbm.at[0, pl.ds(0, CHUNK)], ea,
                              ea_sem).wait()

    def wait_scatter(ob):
        pltpu.make_async_copy(ob, acc.at[slab.at[0, 1]], sc_sem).wait()

    def super_blk(s_i, _):
        sup_base = cid0 + s_i * SUPER

        pltpu.sync_copy(pack_hbm.at[pl.ds(sup_base, SUPER)], slab)

        def issue_chunk(j, xr, ea):
            @pl.when(c == 0)
            def _():
                pltpu.async_copy(xa_hbm.at[slab.at[j, 0]], xr, g_sem)

            @pl.when(c == 1)
            def _():
                pltpu.async_copy(xb_hbm.at[slab.at[j, 0]], xr, g_sem)

            pltpu.async_copy(
                eas_hbm.at[c, pl.ds((sup_base + j) * CHUNK, CHUNK)],
                ea, ea_sem)

        def phase(j, cur_xr, cur_ea, cur_ob, nxt_xr, nxt_ea, drain_scatter):
            jnext = jnp.where(j + 1 == SUPER, 0, j + 1)
            issue_chunk(jnext, nxt_xr, nxt_ea)
            wait_gather(cur_xr)
            wait_ea(cur_ea)

            @pl.when(drain_scatter)
            def _():
                wait_scatter(cur_ob)

            for e in range(CHUNK):
                for p in range(HALF // 32):
                    w = plsc.bitcast(cur_ea[e, pl.ds(32 * p, 32)], jnp.int32)
                    eva = plsc.bitcast(jnp.left_shift(w, 16), jnp.float32)
                    evb = plsc.bitcast(
                        jnp.bitwise_and(w, jnp.int32(-65536)), jnp.float32)
                    for k, ev in ((2 * p, eva), (2 * p + 1, evb)):
                        xv = cur_xr[e, pl.ds(16 * k, 16)]
                        m = jnp.maximum(xv + ev, EPS)
                        t = jnp.exp(m)
                        cur_ob[e, pl.ds(16 * k, 16)] = t * m
                        cur_ob[e, pl.ds(HALF + 16 * k, 16)] = t
            pltpu.async_copy(cur_ob, acc.at[slab.at[j, 1]], sc_sem, add=True)

        issue_chunk(jnp.int32(0), xr0, ea0)

        def step(j2, _):
            j = 2 * j2
            phase(j, xr0, ea0, ob0, xr1, ea1, j2 > 0)
            phase(j + 1, xr1, ea1, ob1, xr0, ea0, j2 > 0)
            return 0

        lax.fori_loop(0, SUPER // 2, step, 0)

        wait_gather(xr0)
        wait_ea(ea0)
        wait_scatter(ob0)
        wait_scatter(ob1)
        return 0

    lax.fori_loop(0, NCHUNKS // SUPER, super_blk, 0)

    plsc.subcore_barrier()

    pltpu.sync_copy(acc.at[pl.ds(r0, ROWS_PER_TILE)],
                    out_hbm.at[c, pl.ds(r0, ROWS_PER_TILE)])

    @pl.when(s == NSUB - 1)
    def _():
        pltpu.sync_copy(acc.at[pl.ds(NSUB * ROWS_PER_TILE, ROWS_TAIL)],
                        out_hbm.at[c, pl.ds(NSUB * ROWS_PER_TILE, ROWS_TAIL)])


def _sc_edge_pass(xa, xb, eas, pack, zeros):
    mesh = plsc.VectorSubcoreMesh(core_axis_name="c", subcore_axis_name="s")
    f = pl.kernel(
        _sc_edge_body,
        out_type=jax.ShapeDtypeStruct((NCORE, N_NODES, 2 * HALF), jnp.float32),
        mesh=mesh,
        compiler_params=pltpu.CompilerParams(use_tc_tiling_on_sc=False,
                                             needs_layout_passes=False),
        scratch_types=[
            pltpu.VMEM_SHARED((N_NODES, 2 * HALF), jnp.float32),
            pltpu.VMEM((SUPER, 2, CHUNK), jnp.int32),
            pltpu.VMEM((CHUNK, HALF), jnp.float32),
            pltpu.VMEM((CHUNK, HALF), jnp.float32),
            pltpu.VMEM((CHUNK, HALF), jnp.bfloat16),
            pltpu.VMEM((CHUNK, HALF), jnp.bfloat16),
            pltpu.VMEM((CHUNK, 2 * HALF), jnp.float32),
            pltpu.VMEM((CHUNK, 2 * HALF), jnp.float32),
            pltpu.SemaphoreType.DMA,
            pltpu.SemaphoreType.DMA,
            pltpu.SemaphoreType.DMA,
        ],
    )
    return f(xa, xb, eas, pack, zeros)


def _update_body(sc_ref, xs_ref, w1_ref, b1_ref, w2_ref, b2_ref,
                 lw_ref, lb_ref, o_ref):
    num = jnp.concatenate([sc_ref[0, :, :HALF], sc_ref[1, :, :HALF]], axis=1)
    den = jnp.concatenate([sc_ref[0, :, HALF:], sc_ref[1, :, HALF:]], axis=1)
    x = jnp.concatenate([xs_ref[0], xs_ref[1]], axis=1)
    h = num / (den + 1e-16) + x
    h = jnp.maximum(jnp.dot(h, w1_ref[...],
                            preferred_element_type=jnp.float32) + b1_ref[...],
                    0.0)
    h = jnp.dot(h, w2_ref[...], preferred_element_type=jnp.float32) + b2_ref[...]
    y = jnp.maximum(jnp.dot(h, lw_ref[...],
                            preferred_element_type=jnp.float32) + lb_ref[...],
                    0.0)
    o_ref[0] = y[:, :HALF]
    o_ref[1] = y[:, HALF:]


def _node_update(sc_out, xs, gp, lp):
    bn = 2000
    grid = (N_NODES // bn,)
    full = lambda a: pl.BlockSpec(a.shape, lambda n: (0,) * a.ndim)
    w1 = gp["W1"]
    b1 = gp["b1"].reshape(1, -1)
    w2 = gp["W2"]
    b2 = gp["b2"].reshape(1, -1)
    lw = lp["W"]
    lb = lp["b"].reshape(1, -1)
    return pl.pallas_call(
        _update_body,
        grid=grid,
        in_specs=[
            pl.BlockSpec((NCORE, bn, 2 * HALF), lambda n: (0, n, 0)),
            pl.BlockSpec((NCORE, bn, HALF), lambda n: (0, n, 0)),
            full(w1), full(b1), full(w2), full(b2), full(lw), full(lb),
        ],
        out_specs=pl.BlockSpec((NCORE, bn, HALF), lambda n: (0, n, 0)),
        out_shape=jax.ShapeDtypeStruct((NCORE, N_NODES, HALF), jnp.float32),
    )(sc_out, xs, w1, b1, w2, b2, lw, lb)


def _update_pool_body(sc_ref, xs_ref, w1_ref, b1_ref, w2_ref, b2_ref,
                      lw_ref, lb_ref, bi_ref, sums_ref, cnt_ref):
    @pl.when(pl.program_id(0) == 0)
    def _():
        sums_ref[...] = jnp.zeros_like(sums_ref)
        cnt_ref[...] = jnp.zeros_like(cnt_ref)

    num = jnp.concatenate([sc_ref[0, :, :HALF], sc_ref[1, :, :HALF]], axis=1)
    den = jnp.concatenate([sc_ref[0, :, HALF:], sc_ref[1, :, HALF:]], axis=1)
    x = jnp.concatenate([xs_ref[0], xs_ref[1]], axis=1)
    h = num / (den + 1e-16) + x
    h = jnp.maximum(jnp.dot(h, w1_ref[...],
                            preferred_element_type=jnp.float32) + b1_ref[...],
                    0.0)
    h = jnp.dot(h, w2_ref[...], preferred_element_type=jnp.float32) + b2_ref[...]
    y = jnp.maximum(jnp.dot(h, lw_ref[...],
                            preferred_element_type=jnp.float32) + lb_ref[...],
                    0.0)
    bi = bi_ref[0, 0, :]
    gids = lax.broadcasted_iota(jnp.int32, (N_GRAPHS, bi.shape[0]), 0)
    mask = (bi[None, :] == gids).astype(jnp.float32)
    sums_ref[...] += jnp.dot(mask, y, preferred_element_type=jnp.float32)
    cnt_ref[...] += jnp.sum(mask, axis=1, keepdims=True)


def _update_pool(sc_out, xs, gp, lp, batch_index):
    bn = 2000
    grid = (N_NODES // bn,)
    full = lambda a: pl.BlockSpec(a.shape, lambda n: (0,) * a.ndim)
    w1 = gp["W1"]
    b1 = gp["b1"].reshape(1, -1)
    w2 = gp["W2"]
    b2 = gp["b2"].reshape(1, -1)
    lw = lp["W"]
    lb = lp["b"].reshape(1, -1)
    bi3 = batch_index.reshape(N_NODES // bn, 1, bn)
    return pl.pallas_call(
        _update_pool_body,
        grid=grid,
        in_specs=[
            pl.BlockSpec((NCORE, bn, 2 * HALF), lambda n: (0, n, 0)),
            pl.BlockSpec((NCORE, bn, HALF), lambda n: (0, n, 0)),
            full(w1), full(b1), full(w2), full(b2), full(lw), full(lb),
            pl.BlockSpec((1, 1, bn), lambda n: (n, 0, 0)),
        ],
        out_specs=[
            pl.BlockSpec((N_GRAPHS, D_FEAT), lambda n: (0, 0)),
            pl.BlockSpec((N_GRAPHS, 1), lambda n: (0, 0)),
        ],
        out_shape=[
            jax.ShapeDtypeStruct((N_GRAPHS, D_FEAT), jnp.float32),
            jax.ShapeDtypeStruct((N_GRAPHS, 1), jnp.float32),
        ],
    )(sc_out, xs, w1, b1, w2, b2, lw, lb, bi3)


def _head_body(sums_ref, cnt_ref, mol_ref,
               m0w, m0b, m1w, m1b, m2w, m2b,
               p0w, p0b, p1w, p1b, ow, ob, o_ref):
    h1 = sums_ref[...] / jnp.maximum(cnt_ref[...], 1.0)
    h2 = mol_ref[...]
    for w, b in ((m0w, m0b), (m1w, m1b), (m2w, m2b)):
        h2 = jnp.maximum(jnp.dot(h2, w[...],
                                 preferred_element_type=jnp.float32) + b[...],
                         0.0)
    w0 = p0w[...]
    h = jnp.maximum(jnp.dot(h1, w0[:D_FEAT],
                            preferred_element_type=jnp.float32)
                    + jnp.dot(h2, w0[D_FEAT:],
                              preferred_element_type=jnp.float32)
                    + p0b[...], 0.0)
    h = jnp.maximum(jnp.dot(h, p1w[...],
                            preferred_element_type=jnp.float32) + p1b[...],
                    0.0)
    o_ref[...] = jnp.dot(h, ow[...],
                         preferred_element_type=jnp.float32) + ob[...]


def _head(sums, cnt, mol_features, params):
    mlp = params["mlp"]
    pred = params["pred"]
    out = params["out"]
    args = [sums, cnt, mol_features,
            mlp[0]["W"], mlp[0]["b"].reshape(1, -1),
            mlp[1]["W"], mlp[1]["b"].reshape(1, -1),
            mlp[2]["W"], mlp[2]["b"].reshape(1, -1),
            pred[0]["W"], pred[0]["b"].reshape(1, -1),
            pred[1]["W"], pred[1]["b"].reshape(1, -1),
            out["W"], out["b"].reshape(1, -1)]
    full = lambda a: pl.BlockSpec(a.shape, lambda: (0,) * a.ndim)
    return pl.pallas_call(
        _head_body,
        in_specs=[full(a) for a in args],
        out_specs=pl.BlockSpec((N_GRAPHS, 1), lambda: (0, 0)),
        out_shape=jax.ShapeDtypeStruct((N_GRAPHS, 1), jnp.float32),
    )(*args)


def kernel(x, edge_index, edge_attr, batch_index, mol_features, params):
    pack = jnp.concatenate(
        [edge_index[0].reshape(N_EDGES // CHUNK, 1, CHUNK),
         edge_index[1].reshape(N_EDGES // CHUNK, 1, CHUNK)], axis=1)

    wcat = jnp.concatenate([p["We"] for p in params["gcn"]], axis=1)
    bcat = jnp.concatenate([p["be"] for p in params["gcn"]]).reshape(1, -1)
    perm_half = []
    for blk in range(HALF // 32):
        for i in range(16):
            perm_half += [32 * blk + i, 32 * blk + 16 + i]
    perm_layer = perm_half + [HALF + v for v in perm_half]
    full_perm = jnp.array([l * D_FEAT + v for l in range(3)
                           for v in perm_layer], dtype=jnp.int32)
    eas = _edge_linears(edge_attr, wcat[:, full_perm], bcat[:, full_perm])

    zeros = jnp.zeros((ROWS_PER_TILE, 2 * HALF), jnp.float32)
    xs = jnp.stack([x[:, :HALF], x[:, HALF:]])
    sc1 = _sc_edge_pass(xs[0], xs[1], eas[0], pack, zeros)
    xs1 = _node_update(sc1, xs, params["gcn"][0], params["lin"][0])
    sc2 = _sc_edge_pass(xs1[0], xs1[1], eas[1], pack, zeros)
    xs2 = _node_update(sc2, xs1, params["gcn"][1], params["lin"][1])
    sc3 = _sc_edge_pass(xs2[0], xs2[1], eas[2], pack, zeros)
    sums, cnt = _update_pool(sc3, xs2, params["gcn"][2], params["lin"][2],
                             batch_index)
    return _head(sums, cnt, mol_features, params)
